# baseline (speedup 1.0000x reference)
import functools

import jax
import jax.numpy as jnp
from jax import lax
from jax.experimental import pallas as pl
from jax.experimental.pallas import tpu as pltpu

_TF_ROTS = ((13, 15, 26, 6), (17, 29, 16, 24))


def _threefry2x32(x0, x1):
    ks = (jnp.uint32(0), jnp.uint32(42), jnp.uint32(0 ^ 42 ^ 0x1BD11BDA))
    x0 = x0 + ks[0]
    x1 = x1 + ks[1]
    for i in range(5):
        for r in _TF_ROTS[i % 2]:
            x0 = x0 + x1
            x1 = (x1 << r) | (x1 >> (32 - r))
            x1 = x0 ^ x1
        x0 = x0 + ks[(i + 1) % 3]
        x1 = x1 + ks[(i + 2) % 3] + jnp.uint32(i + 1)
    return x0, x1


def _body(x_ref, o_ref, acc_v, acc_i, *, R, ROWS, CT, C, NC):
    b = pl.program_id(0)
    k = pl.program_id(1)

    @pl.when(k == 0)
    def _init():
        acc_v[...] = jnp.full((1, 128), -jnp.inf, jnp.float32)
        acc_i[...] = jnp.zeros((1, 128), jnp.int32)

    rows = lax.broadcasted_iota(jnp.int32, (R, 128), 0)
    lanes = lax.broadcasted_iota(jnp.int32, (R, 128), 1)
    gr = k * R + rows
    t = lanes & 15
    grp = lanes >> 4
    c = gr * 8 + grp
    idx = b * CT + t * C + c

    x1 = idx.astype(jnp.uint32)
    x0 = jnp.zeros_like(x1)
    o0, o1 = _threefry2x32(x0, x1)
    bits = o0 ^ o1

    fb = (bits >> 9) | jnp.uint32(0x3F800000)
    f = lax.bitcast_convert_type(fb, jnp.float32) - jnp.float32(1.0)
    u = jnp.maximum(f, jnp.float32(1.1754944e-38))
    g = -jnp.log(-jnp.log(u))

    val = x_ref[0] + g
    val = jnp.where(gr < ROWS, val, -jnp.inf)

    m = jnp.max(val, axis=0, keepdims=True)
    hit = jnp.where(val == m, rows, jnp.int32(2**30))
    fr = jnp.min(hit, axis=0, keepdims=True)
    cand_c = (k * R + fr) * 8 + (lanes[:1] >> 4)

    better = m > acc_v[...]
    acc_v[...] = jnp.where(better, m, acc_v[...])
    acc_i[...] = jnp.where(better, cand_c, acc_i[...])

    @pl.when(k == NC - 1)
    def _merge():
        bv = acc_v[...]
        bi = acc_i[...]
        best_v = bv[:, 0:16]
        best_i = bi[:, 0:16]
        for gidx in range(1, 8):
            vv = bv[:, gidx * 16:(gidx + 1) * 16]
            ii = bi[:, gidx * 16:(gidx + 1) * 16]
            take = (vv > best_v) | ((vv == best_v) & (ii < best_i))
            best_v = jnp.where(take, vv, best_v)
            best_i = jnp.where(take, ii, best_i)
        o_ref[0, 0, :] = best_i[0]


def _sampler(Xr, *, B, C, T, R, ROWS, NC, interpret=False):
    return pl.pallas_call(
        functools.partial(_body, R=R, ROWS=ROWS, CT=C * T, C=C, NC=NC),
        grid=(B, NC),
        in_specs=[pl.BlockSpec((1, R, 128), lambda b, k: (b, k, 0))],
        out_specs=pl.BlockSpec((1, 1, 16), lambda b, k: (b, 0, 0)),
        out_shape=jax.ShapeDtypeStruct((B, 1, 16), jnp.int32),
        scratch_shapes=[pltpu.VMEM((1, 128), jnp.float32),
                        pltpu.VMEM((1, 128), jnp.int32)],
        compiler_params=pltpu.CompilerParams(
            dimension_semantics=("parallel", "arbitrary")),
        interpret=interpret,
    )(Xr)


def kernel(X, interpret=False):
    if X.ndim == 2:
        X = X[None]
    B, C, T = X.shape
    CT = C * T
    assert T == 16 and CT % 128 == 0, (B, C, T)
    ROWS = CT // 128
    R = min(512, ROWS)
    NC = (ROWS + R - 1) // R
    Xr = X.reshape(B, ROWS, 128)
    out = _sampler(Xr, B=B, C=C, T=T, R=R, ROWS=ROWS, NC=NC,
                   interpret=interpret)
    return out.reshape(B, T)
